# baseline (speedup 1.0000x reference)
import functools

import jax
import jax.numpy as jnp
from jax import lax
from jax.experimental import pallas as pl
from jax.experimental.pallas import tpu as pltpu
from jax.experimental.pallas import tpu_sc as plsc

N = 10000
DIN = 128
FEAT = 128
ACCW = 144
NTILE = 16
NSC = 2
NPAD = 10240
ANROWS = 10016
ZROWS = 626
OROWS = 625
EDGE_K = 56
NCHUNK = 186
PER_TILE = EDGE_K * NCHUNK
EPAD = PER_TILE * NTILE * NSC
TOTCHUNK = NCHUNK * NTILE * NSC
TCB = 512


def _make_edge_kernel(heads):
    mesh = plsc.VectorSubcoreMesh(core_axis_name="c", subcore_axis_name="s")

    @functools.partial(
        pl.kernel, mesh=mesh,
        compiler_params=pltpu.CompilerParams(use_tc_tiling_on_sc=False),
        out_type=jax.ShapeDtypeStruct((NSC, NPAD, ACCW), jnp.float32),
        scratch_types=[
            pltpu.VMEM((3, 2, EDGE_K), jnp.int32),
            pltpu.VMEM((2, EDGE_K, FEAT), jnp.float32),
            pltpu.VMEM((2, EDGE_K, FEAT), jnp.float32),
            pltpu.VMEM((EDGE_K, ACCW), jnp.float32),
            pltpu.VMEM((FEAT,), jnp.float32),
            pltpu.VMEM_SHARED((ANROWS, ACCW), jnp.float32),
            pltpu.SemaphoreType.DMA((2,)),
            pltpu.SemaphoreType.DMA((2,)),
        ],
    )
    def edge_kernel(xl_hbm, xr_hbm, edges_hbm, att_hbm, out_hbm,
                    eb_v, g1_v, g2_v, con_v, att_v, acc_sh,
                    semi, semg):
        cid = lax.axis_index("c")
        tid = lax.axis_index("s")
        wid = cid * NTILE + tid

        def zrow(r, carry):
            for j in range(ACCW // 16):
                con_v[r, pl.ds(j * 16, 16)] = jnp.zeros((16,), jnp.float32)
            return carry
        lax.fori_loop(0, EDGE_K, zrow, 0)
        zrow0 = tid * ZROWS
        for b in range(ZROWS // EDGE_K):
            pltpu.sync_copy(con_v,
                            acc_sh.at[pl.ds(zrow0 + b * EDGE_K, EDGE_K)])
        ztail = ZROWS % EDGE_K
        pltpu.sync_copy(con_v.at[pl.ds(0, ztail)],
                        acc_sh.at[pl.ds(zrow0 + ZROWS - ztail, ztail)])
        plsc.subcore_barrier()

        pltpu.sync_copy(att_hbm, att_v)
        att = [att_v[pl.ds(j * 16, 16)] for j in range(8)]
        lane = lax.iota(jnp.int32, 16)
        onehot = [jnp.where(lane == h, 1.0, 0.0).astype(jnp.float32)
                  for h in range(heads)]
        perms = [(lane ^ (1 << k)).reshape(16, 1) for k in range(4)]
        dnums = lax.GatherDimensionNumbers(
            offset_dims=(), collapsed_slice_dims=(0,), start_index_map=(0,))

        def G(x, p):
            return lax.gather(x, p, dnums, (1,),
                              mode=lax.GatherScatterMode.PROMISE_IN_BOUNDS)

        def vsum_bcast(t):
            for p in perms:
                t = t + G(t, p)
            return t

        lmask8 = lane < 8
        lmask4 = (lane & 4) == 0
        lmask2 = (lane & 2) == 0
        hpos = [2 * (((h & 1) << 2) | (h & 2) | ((h >> 2) & 1))
                for h in range(8)]
        hsplat = [jnp.full((16, 1), p, jnp.int32) for p in hpos]
        wvperm = ((((lane & 1) * 4) + (lane & 2) + ((lane // 4) & 1))
                  * 2).reshape(16, 1)
        mask8f = jnp.where(lmask8, 1.0, 0.0).astype(jnp.float32)

        cbase = wid * NCHUNK

        def start_idx(c):
            pltpu.async_copy(edges_hbm.at[cbase + c], eb_v.at[lax.rem(c, 3)],
                             semi.at[lax.rem(c, 2)])

        def wait_idx(c):
            pltpu.make_async_copy(edges_hbm.at[cbase + c],
                                  eb_v.at[lax.rem(c, 3)],
                                  semi.at[lax.rem(c, 2)]).wait()

        def start_gathers(c):
            s2, s3 = lax.rem(c, 2), lax.rem(c, 3)
            pltpu.async_copy(xl_hbm.at[eb_v.at[s3, 0]], g1_v.at[s2], semg.at[s2])
            pltpu.async_copy(xr_hbm.at[eb_v.at[s3, 1]], g2_v.at[s2], semg.at[s2])

        def wait_gathers(c):
            s2, s3 = lax.rem(c, 2), lax.rem(c, 3)
            pltpu.make_async_copy(xl_hbm.at[eb_v.at[s3, 0]], g1_v.at[s2],
                                  semg.at[s2]).wait()
            pltpu.make_async_copy(xr_hbm.at[eb_v.at[s3, 1]], g2_v.at[s2],
                                  semg.at[s2]).wait()

        def run_scatter(c):
            s3 = lax.rem(c, 3)
            pltpu.sync_copy(con_v, acc_sh.at[eb_v.at[s3, 1]], add=True)

        pltpu.sync_copy(edges_hbm.at[cbase + 0], eb_v.at[0])
        start_gathers(0)
        start_idx(1)
        start_idx(2)

        def chunk_body(c, carry):
            cur = lax.rem(c, 2)

            @pl.when(c + 1 < NCHUNK)
            def _():
                wait_idx(c + 1)
                start_gathers(c + 1)

            @pl.when(jnp.logical_and(c >= 1, c + 2 < NCHUNK))
            def _():
                start_idx(c + 2)

            wait_gathers(c)

            @plsc.parallel_loop(0, EDGE_K, unroll=4)
            def edge_body(e):
                g1 = [g1_v[cur, e, pl.ds(j * 16, 16)] for j in range(8)]
                g2 = [g2_v[cur, e, pl.ds(j * 16, 16)] for j in range(8)]
                t = []
                for j in range(8):
                    m = g1[j] + g2[j]
                    t.append(jnp.maximum(m, 0.2 * m) * att[j])
                wvec = jnp.zeros((16,), jnp.float32)
                if heads == 8:
                    for h in range(8):
                        wb = jnp.exp(vsum_bcast(t[h]))
                        wvec = wvec + wb * onehot[h]
                        con_v[e, pl.ds(h * 16, 16)] = g1[h] * wb
                else:
                    while len(t) > 1:
                        t = [a + b for a, b in zip(t[::2], t[1::2])]
                    wb = jnp.exp(vsum_bcast(t[0]))
                    wvec = wb * onehot[0]
                    for j in range(8):
                        con_v[e, pl.ds(j * 16, 16)] = g1[j] * wb
                con_v[e, pl.ds(128, 16)] = wvec

            run_scatter(c)
            return carry

        lax.fori_loop(0, NCHUNK, chunk_body, 0)
        plsc.subcore_barrier()

        orow0 = tid * OROWS
        pltpu.sync_copy(acc_sh.at[pl.ds(orow0, OROWS)],
                        out_hbm.at[cid, pl.ds(orow0, OROWS)])

    return edge_kernel


_edge_l1 = _make_edge_kernel(8)
_edge_l2 = _make_edge_kernel(1)


def _mm2_body(x_ref, wl_ref, wr_ref, ol_ref, or_ref):
    xb = x_ref[...]
    ol_ref[...] = jnp.dot(xb, wl_ref[...], preferred_element_type=jnp.float32)
    or_ref[...] = jnp.dot(xb, wr_ref[...], preferred_element_type=jnp.float32)


def _mm2(x_pad, wl, wr):
    return pl.pallas_call(
        _mm2_body,
        grid=(NPAD // TCB,),
        in_specs=[pl.BlockSpec((TCB, DIN), lambda i: (i, 0)),
                  pl.BlockSpec((DIN, FEAT), lambda i: (0, 0)),
                  pl.BlockSpec((DIN, FEAT), lambda i: (0, 0))],
        out_specs=[pl.BlockSpec((TCB, FEAT), lambda i: (i, 0)),
                   pl.BlockSpec((TCB, FEAT), lambda i: (i, 0))],
        out_shape=[jax.ShapeDtypeStruct((NPAD, FEAT), jnp.float32)] * 2,
    )(x_pad, wl, wr)


def _combine_mm_body(a0_ref, a1_ref, b_ref, wl_ref, wr_ref, ol_ref, or_ref):
    a = a0_ref[...] + a1_ref[...]
    num = a[:, :FEAT]
    den8 = a[:, FEAT:FEAT + 8]
    col = lax.broadcasted_iota(jnp.int32, (8, FEAT), 1)
    row = lax.broadcasted_iota(jnp.int32, (8, FEAT), 0)
    expand = jnp.where(col // 16 == row, 1.0, 0.0).astype(jnp.float32)
    den = jnp.dot(den8, expand, preferred_element_type=jnp.float32) + 1e-16
    h = jnp.maximum(num / den + b_ref[...], 0.0)
    ol_ref[...] = jnp.dot(h, wl_ref[...], preferred_element_type=jnp.float32)
    or_ref[...] = jnp.dot(h, wr_ref[...], preferred_element_type=jnp.float32)


def _combine_mm(a0, a1, b1, wl, wr):
    return pl.pallas_call(
        _combine_mm_body,
        grid=(NPAD // TCB,),
        in_specs=[pl.BlockSpec((TCB, ACCW), lambda i: (i, 0)),
                  pl.BlockSpec((TCB, ACCW), lambda i: (i, 0)),
                  pl.BlockSpec((1, FEAT), lambda i: (0, 0)),
                  pl.BlockSpec((FEAT, FEAT), lambda i: (0, 0)),
                  pl.BlockSpec((FEAT, FEAT), lambda i: (0, 0))],
        out_specs=[pl.BlockSpec((TCB, FEAT), lambda i: (i, 0)),
                   pl.BlockSpec((TCB, FEAT), lambda i: (i, 0))],
        out_shape=[jax.ShapeDtypeStruct((NPAD, FEAT), jnp.float32)] * 2,
    )(a0, a1, b1, wl, wr)


def _final_body(a0_ref, a1_ref, b_ref, o_ref):
    a = a0_ref[...] + a1_ref[...]
    den = a[:, FEAT:FEAT + 1] + 1e-16
    o_ref[...] = jnp.maximum(a[:, :FEAT] / den + b_ref[...], 0.0)


def _final(a0, a1, b2):
    return pl.pallas_call(
        _final_body,
        grid=(NPAD // TCB,),
        in_specs=[pl.BlockSpec((TCB, ACCW), lambda i: (i, 0)),
                  pl.BlockSpec((TCB, ACCW), lambda i: (i, 0)),
                  pl.BlockSpec((1, FEAT), lambda i: (0, 0))],
        out_specs=pl.BlockSpec((TCB, FEAT), lambda i: (i, 0)),
        out_shape=jax.ShapeDtypeStruct((NPAD, FEAT), jnp.float32),
    )(a0, a1, b2)


def kernel(x, edge_index, Wl1, Wr1, att1, b1, Wl2, Wr2, att2, b2):
    E = edge_index.shape[1]
    loops = jnp.arange(N, dtype=jnp.int32)
    src = jnp.concatenate([edge_index[0].astype(jnp.int32), loops])
    dst = jnp.concatenate([edge_index[1].astype(jnp.int32), loops])
    npad_e = EPAD - (E + N)
    pad_rows = N + (jnp.arange(npad_e, dtype=jnp.int32) % 8)
    src = jnp.concatenate([src, pad_rows])
    dst = jnp.concatenate([dst, pad_rows])
    edges = jnp.stack([src.reshape(TOTCHUNK, EDGE_K),
                       dst.reshape(TOTCHUNK, EDGE_K)], axis=1)

    x_pad = jnp.zeros((NPAD, DIN), jnp.float32).at[:N].set(x)
    xl1, xr1 = _mm2(x_pad, Wl1, Wr1)
    acc1 = _edge_l1(xl1, xr1, edges, att1.reshape(FEAT))
    xl2, xr2 = _combine_mm(acc1[0], acc1[1], b1.reshape(1, FEAT), Wl2, Wr2)
    acc2 = _edge_l2(xl2, xr2, edges, att2.reshape(FEAT))
    out_pad = _final(acc2[0], acc2[1], b2.reshape(1, FEAT))
    return out_pad[:N]


# speedup vs baseline: 1.0007x; 1.0007x over previous
import functools

import jax
import jax.numpy as jnp
from jax import lax
from jax.experimental import pallas as pl
from jax.experimental.pallas import tpu as pltpu
from jax.experimental.pallas import tpu_sc as plsc

N = 10000
DIN = 128
FEAT = 128
ACCW = 144
NTILE = 16
NSC = 2
NPAD = 10240
ANROWS = 10016
ZROWS = 626
OROWS = 625
EDGE_K = 56
NCHUNK = 186
PER_TILE = EDGE_K * NCHUNK
EPAD = PER_TILE * NTILE * NSC
TOTCHUNK = NCHUNK * NTILE * NSC
TCB = 512


def _make_edge_kernel(heads):
    mesh = plsc.VectorSubcoreMesh(core_axis_name="c", subcore_axis_name="s")

    @functools.partial(
        pl.kernel, mesh=mesh,
        compiler_params=pltpu.CompilerParams(use_tc_tiling_on_sc=False),
        out_type=jax.ShapeDtypeStruct((NSC, NPAD, ACCW), jnp.float32),
        scratch_types=[
            pltpu.VMEM((3, 2, EDGE_K), jnp.int32),
            pltpu.VMEM((2, EDGE_K, FEAT), jnp.float32),
            pltpu.VMEM((2, EDGE_K, FEAT), jnp.float32),
            pltpu.VMEM((EDGE_K, ACCW), jnp.float32),
            pltpu.VMEM((FEAT,), jnp.float32),
            pltpu.VMEM_SHARED((ANROWS, ACCW), jnp.float32),
            pltpu.SemaphoreType.DMA((2,)),
            pltpu.SemaphoreType.DMA((2,)),
        ],
    )
    def edge_kernel(xl_hbm, xr_hbm, edges_hbm, att_hbm, out_hbm,
                    eb_v, g1_v, g2_v, con_v, att_v, acc_sh,
                    semi, semg):
        cid = lax.axis_index("c")
        tid = lax.axis_index("s")
        wid = cid * NTILE + tid

        def zrow(r, carry):
            for j in range(ACCW // 16):
                con_v[r, pl.ds(j * 16, 16)] = jnp.zeros((16,), jnp.float32)
            return carry
        lax.fori_loop(0, EDGE_K, zrow, 0)
        zrow0 = tid * ZROWS
        for b in range(ZROWS // EDGE_K):
            pltpu.sync_copy(con_v,
                            acc_sh.at[pl.ds(zrow0 + b * EDGE_K, EDGE_K)])
        ztail = ZROWS % EDGE_K
        pltpu.sync_copy(con_v.at[pl.ds(0, ztail)],
                        acc_sh.at[pl.ds(zrow0 + ZROWS - ztail, ztail)])
        plsc.subcore_barrier()

        pltpu.sync_copy(att_hbm, att_v)
        att = [att_v[pl.ds(j * 16, 16)] for j in range(8)]
        lane = lax.iota(jnp.int32, 16)
        onehot = [jnp.where(lane == h, 1.0, 0.0).astype(jnp.float32)
                  for h in range(heads)]
        perms = [(lane ^ (1 << k)).reshape(16, 1) for k in range(4)]
        dnums = lax.GatherDimensionNumbers(
            offset_dims=(), collapsed_slice_dims=(0,), start_index_map=(0,))

        def G(x, p):
            return lax.gather(x, p, dnums, (1,),
                              mode=lax.GatherScatterMode.PROMISE_IN_BOUNDS)

        def vsum_bcast(t):
            for p in perms:
                t = t + G(t, p)
            return t

        cbase = wid * NCHUNK

        def start_idx(c):
            pltpu.async_copy(edges_hbm.at[cbase + c], eb_v.at[lax.rem(c, 3)],
                             semi.at[lax.rem(c, 2)])

        def wait_idx(c):
            pltpu.make_async_copy(edges_hbm.at[cbase + c],
                                  eb_v.at[lax.rem(c, 3)],
                                  semi.at[lax.rem(c, 2)]).wait()

        def start_gathers(c):
            s2, s3 = lax.rem(c, 2), lax.rem(c, 3)
            pltpu.async_copy(xl_hbm.at[eb_v.at[s3, 0]], g1_v.at[s2], semg.at[s2])
            pltpu.async_copy(xr_hbm.at[eb_v.at[s3, 1]], g2_v.at[s2], semg.at[s2])

        def wait_gathers(c):
            s2, s3 = lax.rem(c, 2), lax.rem(c, 3)
            pltpu.make_async_copy(xl_hbm.at[eb_v.at[s3, 0]], g1_v.at[s2],
                                  semg.at[s2]).wait()
            pltpu.make_async_copy(xr_hbm.at[eb_v.at[s3, 1]], g2_v.at[s2],
                                  semg.at[s2]).wait()

        def run_scatter(c):
            s3 = lax.rem(c, 3)
            pltpu.sync_copy(con_v, acc_sh.at[eb_v.at[s3, 1]], add=True)

        pltpu.sync_copy(edges_hbm.at[cbase + 0], eb_v.at[0])
        start_gathers(0)
        start_idx(1)
        start_idx(2)

        def chunk_body(c, carry):
            cur = lax.rem(c, 2)

            @pl.when(c + 1 < NCHUNK)
            def _():
                wait_idx(c + 1)
                start_gathers(c + 1)

            @pl.when(jnp.logical_and(c >= 1, c + 2 < NCHUNK))
            def _():
                start_idx(c + 2)

            wait_gathers(c)

            @plsc.parallel_loop(0, EDGE_K, unroll=4)
            def edge_body(e):
                g1 = [g1_v[cur, e, pl.ds(j * 16, 16)] for j in range(8)]
                g2 = [g2_v[cur, e, pl.ds(j * 16, 16)] for j in range(8)]
                t = []
                for j in range(8):
                    m = g1[j] + g2[j]
                    t.append(jnp.maximum(m, 0.2 * m) * att[j])
                wvec = jnp.zeros((16,), jnp.float32)
                if heads == 8:
                    for h in range(8):
                        wb = jnp.exp(vsum_bcast(t[h]))
                        wvec = wvec + wb * onehot[h]
                        con_v[e, pl.ds(h * 16, 16)] = g1[h] * wb
                else:
                    while len(t) > 1:
                        t = [a + b for a, b in zip(t[::2], t[1::2])]
                    wb = jnp.exp(vsum_bcast(t[0]))
                    wvec = wb * onehot[0]
                    for j in range(8):
                        con_v[e, pl.ds(j * 16, 16)] = g1[j] * wb
                con_v[e, pl.ds(128, 16)] = wvec

            run_scatter(c)
            return carry

        lax.fori_loop(0, NCHUNK, chunk_body, 0)
        plsc.subcore_barrier()

        orow0 = tid * OROWS
        pltpu.sync_copy(acc_sh.at[pl.ds(orow0, OROWS)],
                        out_hbm.at[cid, pl.ds(orow0, OROWS)])

    return edge_kernel


_edge_l1 = _make_edge_kernel(8)
_edge_l2 = _make_edge_kernel(1)


def _mm2_body(x_ref, wl_ref, wr_ref, ol_ref, or_ref):
    xb = x_ref[...]
    ol_ref[...] = jnp.dot(xb, wl_ref[...], preferred_element_type=jnp.float32)
    or_ref[...] = jnp.dot(xb, wr_ref[...], preferred_element_type=jnp.float32)


def _mm2(x_pad, wl, wr):
    return pl.pallas_call(
        _mm2_body,
        grid=(NPAD // TCB,),
        in_specs=[pl.BlockSpec((TCB, DIN), lambda i: (i, 0)),
                  pl.BlockSpec((DIN, FEAT), lambda i: (0, 0)),
                  pl.BlockSpec((DIN, FEAT), lambda i: (0, 0))],
        out_specs=[pl.BlockSpec((TCB, FEAT), lambda i: (i, 0)),
                   pl.BlockSpec((TCB, FEAT), lambda i: (i, 0))],
        out_shape=[jax.ShapeDtypeStruct((NPAD, FEAT), jnp.float32)] * 2,
    )(x_pad, wl, wr)


def _combine_mm_body(a0_ref, a1_ref, b_ref, wl_ref, wr_ref, ol_ref, or_ref):
    a = a0_ref[...] + a1_ref[...]
    num = a[:, :FEAT]
    den8 = a[:, FEAT:FEAT + 8]
    col = lax.broadcasted_iota(jnp.int32, (8, FEAT), 1)
    row = lax.broadcasted_iota(jnp.int32, (8, FEAT), 0)
    expand = jnp.where(col // 16 == row, 1.0, 0.0).astype(jnp.float32)
    den = jnp.dot(den8, expand, preferred_element_type=jnp.float32) + 1e-16
    h = jnp.maximum(num / den + b_ref[...], 0.0)
    ol_ref[...] = jnp.dot(h, wl_ref[...], preferred_element_type=jnp.float32)
    or_ref[...] = jnp.dot(h, wr_ref[...], preferred_element_type=jnp.float32)


def _combine_mm(a0, a1, b1, wl, wr):
    return pl.pallas_call(
        _combine_mm_body,
        grid=(NPAD // TCB,),
        in_specs=[pl.BlockSpec((TCB, ACCW), lambda i: (i, 0)),
                  pl.BlockSpec((TCB, ACCW), lambda i: (i, 0)),
                  pl.BlockSpec((1, FEAT), lambda i: (0, 0)),
                  pl.BlockSpec((FEAT, FEAT), lambda i: (0, 0)),
                  pl.BlockSpec((FEAT, FEAT), lambda i: (0, 0))],
        out_specs=[pl.BlockSpec((TCB, FEAT), lambda i: (i, 0)),
                   pl.BlockSpec((TCB, FEAT), lambda i: (i, 0))],
        out_shape=[jax.ShapeDtypeStruct((NPAD, FEAT), jnp.float32)] * 2,
    )(a0, a1, b1, wl, wr)


def _final_body(a0_ref, a1_ref, b_ref, o_ref):
    a = a0_ref[...] + a1_ref[...]
    den = a[:, FEAT:FEAT + 1] + 1e-16
    o_ref[...] = jnp.maximum(a[:, :FEAT] / den + b_ref[...], 0.0)


def _final(a0, a1, b2):
    return pl.pallas_call(
        _final_body,
        grid=(NPAD // TCB,),
        in_specs=[pl.BlockSpec((TCB, ACCW), lambda i: (i, 0)),
                  pl.BlockSpec((TCB, ACCW), lambda i: (i, 0)),
                  pl.BlockSpec((1, FEAT), lambda i: (0, 0))],
        out_specs=pl.BlockSpec((TCB, FEAT), lambda i: (i, 0)),
        out_shape=jax.ShapeDtypeStruct((NPAD, FEAT), jnp.float32),
    )(a0, a1, b2)


def kernel(x, edge_index, Wl1, Wr1, att1, b1, Wl2, Wr2, att2, b2):
    E = edge_index.shape[1]
    loops = jnp.arange(N, dtype=jnp.int32)
    src = jnp.concatenate([edge_index[0].astype(jnp.int32), loops])
    dst = jnp.concatenate([edge_index[1].astype(jnp.int32), loops])
    npad_e = EPAD - (E + N)
    pad_rows = N + (jnp.arange(npad_e, dtype=jnp.int32) % 8)
    src = jnp.concatenate([src, pad_rows])
    dst = jnp.concatenate([dst, pad_rows])
    edges = jnp.stack([src.reshape(TOTCHUNK, EDGE_K),
                       dst.reshape(TOTCHUNK, EDGE_K)], axis=1)

    x_pad = jnp.zeros((NPAD, DIN), jnp.float32).at[:N].set(x)
    xl1, xr1 = _mm2(x_pad, Wl1, Wr1)
    acc1 = _edge_l1(xl1, xr1, edges, att1.reshape(FEAT))
    xl2, xr2 = _combine_mm(acc1[0], acc1[1], b1.reshape(1, FEAT), Wl2, Wr2)
    acc2 = _edge_l2(xl2, xr2, edges, att2.reshape(FEAT))
    out_pad = _final(acc2[0], acc2[1], b2.reshape(1, FEAT))
    return out_pad[:N]
